# baseline (speedup 1.0000x reference)
import functools

import jax
import jax.numpy as jnp
from jax import lax
from jax.experimental import pallas as pl
from jax.experimental.pallas import tpu as pltpu
from jax.experimental.pallas import tpu_sc as plsc

N = 10000
E = 160000
DF = 128
DE = 4
OC = 8
G = 256

NC = 2
NS = 16
NW = NC * NS
CH = 128
EH = E // 2
NCHH = EH // CH
CPT = NCHH // NW
EXTRA = NCHH - CPT * NW


def _sc_mesh():
    return plsc.VectorSubcoreMesh(
        core_axis_name="c", subcore_axis_name="s",
        num_cores=NC, num_subcores=NS)


@functools.cache
def _gather_rows_kernel():
    return pl.kernel(
        _gather_body,
        out_type=jax.ShapeDtypeStruct((EH, DF), jnp.float32),
        mesh=_sc_mesh(),
        scratch_types=[
            pltpu.VMEM((CH,), jnp.int32),
            pltpu.VMEM((CH, DF), jnp.float32),
            pltpu.SemaphoreType.DMA,
        ],
    )


def _gather_body(x_hbm, src_hbm, out_hbm, idx_v, rows_v, sem):
    wid = lax.axis_index("s") * NC + lax.axis_index("c")

    def chunk(off):
        pltpu.sync_copy(src_hbm.at[pl.ds(off, CH)], idx_v)
        pltpu.async_copy(x_hbm.at[idx_v], rows_v, sem).wait()
        pltpu.sync_copy(rows_v, out_hbm.at[pl.ds(off, CH)])

    def body(c, carry):
        chunk((wid * CPT + c) * CH)
        return carry

    lax.fori_loop(0, CPT, body, 0)

    @pl.when(wid < EXTRA)
    def _extra():
        chunk((NW * CPT + wid) * CH)


@functools.cache
def _scatter_add_kernel():
    return pl.kernel(
        _scatter_body,
        out_type=jax.ShapeDtypeStruct((2 * N, DF), jnp.float32),
        mesh=_sc_mesh(),
        scratch_types=[
            pltpu.VMEM((1, CH), jnp.int32),
            pltpu.VMEM((CH, DF), jnp.float32),
            pltpu.VMEM_SHARED((N, DF), jnp.float32),
            pltpu.SemaphoreType.DMA,
        ],
    )


def _scatter_body(msg_hbm, dst_hbm, zero_hbm, out_hbm, idx_v, val_v, acc_sh, sem):
    cid = lax.axis_index("c")
    sid = lax.axis_index("s")
    wid = sid * NC + cid
    ZR = N // 10

    @pl.when(sid < 10)
    def _zero():
        pltpu.sync_copy(zero_hbm.at[pl.ds(sid * ZR, ZR)],
                        acc_sh.at[pl.ds(sid * ZR, ZR)])

    plsc.subcore_barrier()

    def chunk(off):
        pltpu.sync_copy(dst_hbm.at[pl.ds(off, CH)], idx_v.at[0])
        pltpu.sync_copy(msg_hbm.at[pl.ds(off, CH)], val_v)
        pltpu.async_copy(val_v, acc_sh.at[idx_v.at[0]], sem, add=True).wait()

    def body(c, carry):
        chunk((wid * CPT + c) * CH)
        return carry

    lax.fori_loop(0, CPT, body, 0)

    @pl.when(wid < EXTRA)
    def _extra():
        chunk((NW * CPT + wid) * CH)

    plsc.subcore_barrier()

    @pl.when(sid < 10)
    def _flush():
        pltpu.sync_copy(acc_sh.at[pl.ds(sid * ZR, ZR)],
                        out_hbm.at[pl.ds(cid * N + sid * ZR, ZR)])


B = 2000


def _msg_body(ea_ref, xs_ref, w_ref, out_ref):
    ea = ea_ref[...]
    xs = xs_ref[...]
    w = w_ref[...]
    lin = jnp.broadcast_to(w[DE:DE + 1, :], (B, OC * DF))
    for d in range(DE):
        lin = lin + ea[:, d:d + 1] * w[d:d + 1, :]
    t = jnp.maximum(lin, 0.0)
    prod = t * jnp.concatenate([xs] * OC, axis=1)
    parts = [jnp.sum(prod[:, o * DF:(o + 1) * DF], axis=1, keepdims=True)
             for o in range(OC)]
    parts.append(jnp.zeros((B, DF - OC), jnp.float32))
    out_ref[...] = jnp.concatenate(parts, axis=1)


def _messages(edge_attr_aug, xs, W1b):
    return pl.pallas_call(
        _msg_body,
        grid=(EH // B,),
        in_specs=[
            pl.BlockSpec((B, DE + 1), lambda i: (i, 0)),
            pl.BlockSpec((B, DF), lambda i: (i, 0)),
            pl.BlockSpec((DE + 1, OC * DF), lambda i: (0, 0)),
        ],
        out_specs=pl.BlockSpec((B, DF), lambda i: (i, 0)),
        out_shape=jax.ShapeDtypeStruct((EH, DF), jnp.float32),
        compiler_params=pltpu.CompilerParams(
            dimension_semantics=("parallel",)),
    )(edge_attr_aug, xs, W1b)


CK = 2000


def _tail_body(a2_ref, b2_ref, x_ref, root_ref, bc_ref, gm_ref, bt_ref,
               batch_ref, wo_ref, bo_ref, out_ref):
    aggr = (a2_ref[0:N, 0:OC] + a2_ref[N:2 * N, 0:OC]
            + b2_ref[0:N, 0:OC] + b2_ref[N:2 * N, 0:OC])
    h = aggr + jnp.dot(x_ref[...], root_ref[...],
                       precision=lax.Precision.HIGHEST,
                       preferred_element_type=jnp.float32) + bc_ref[...]
    h = jnp.maximum(h, 0.0)
    mean = jnp.mean(h, axis=0, keepdims=True)
    var = jnp.mean(h * h, axis=0, keepdims=True) - mean * mean
    hn = gm_ref[...] * (h - mean) * lax.rsqrt(var + 1e-5) + bt_ref[...]
    gids = lax.broadcasted_iota(jnp.int32, (G, 1), 0)
    pooled = jnp.zeros((G, OC), jnp.float32)
    for k in range(N // CK):
        bk = batch_ref[0:1, k * CK:(k + 1) * CK]
        oh = (jnp.broadcast_to(bk, (G, CK)) == gids).astype(jnp.float32)
        pooled = pooled + lax.dot_general(
            oh, hn[k * CK:(k + 1) * CK, :],
            (((1,), (0,)), ((), ())), precision=lax.Precision.HIGHEST,
            preferred_element_type=jnp.float32)
    pr = jnp.maximum(pooled, 0.0)
    out_ref[...] = jnp.sum(pr * wo_ref[...], axis=1, keepdims=True) + bo_ref[...]


def _tail(a2, b2, x, root, bias_conv, gamma, beta, batch_row, W_out_row, b_out):
    return pl.pallas_call(
        _tail_body,
        out_shape=jax.ShapeDtypeStruct((G, 1), jnp.float32),
    )(a2, b2, x, root, bias_conv, gamma, beta, batch_row, W_out_row, b_out)


@jax.jit
def kernel(x, edge_index, edge_attr, batch, W1, b1, root, bias_conv,
           gamma, beta, W_out, b_out):
    src = edge_index[0].astype(jnp.int32)
    dst = edge_index[1].astype(jnp.int32)
    W1p = W1.reshape(DE, DF, OC).transpose(0, 2, 1).reshape(DE, OC * DF)
    b1p = b1.reshape(DF, OC).T.reshape(1, OC * DF)
    W1b = jnp.concatenate([W1p, b1p], axis=0)
    ea_aug = jnp.concatenate(
        [edge_attr, jnp.ones((E, 1), jnp.float32)], axis=1)
    zeros = jnp.zeros((N, DF), jnp.float32)

    gather = _gather_rows_kernel()
    scatter = _scatter_add_kernel()
    xsA = gather(x, src[:EH])
    xsB = gather(x, src[EH:])
    msgA = _messages(ea_aug[:EH], xsA, W1b)
    a2A = scatter(msgA, dst[:EH], zeros)
    msgB = _messages(ea_aug[EH:], xsB, W1b)
    a2B = scatter(msgB, dst[EH:], zeros)
    out = _tail(a2A, a2B, x, root,
                bias_conv.reshape(1, OC), gamma.reshape(1, OC),
                beta.reshape(1, OC),
                batch.astype(jnp.int32).reshape(1, N),
                W_out.reshape(1, OC), b_out.reshape(1, 1))
    return out


# speedup vs baseline: 1.0924x; 1.0924x over previous
import functools

import jax
import jax.numpy as jnp
from jax import lax
from jax.experimental import pallas as pl
from jax.experimental.pallas import tpu as pltpu
from jax.experimental.pallas import tpu_sc as plsc

N = 10000
E = 160000
DF = 128
DE = 4
OC = 8
G = 256

NC = 2
NS = 16
NW = NC * NS
CH = 128
EH = E // 2
NCHH = EH // CH
CPT = NCHH // NW
EXTRA = NCHH - CPT * NW


def _sc_mesh():
    return plsc.VectorSubcoreMesh(
        core_axis_name="c", subcore_axis_name="s",
        num_cores=NC, num_subcores=NS)


@functools.cache
def _gather_rows_kernel():
    return pl.kernel(
        _gather_body,
        out_type=jax.ShapeDtypeStruct((EH, DF), jnp.float32),
        mesh=_sc_mesh(),
        scratch_types=[
            pltpu.VMEM((CH,), jnp.int32),
            pltpu.VMEM((CH, DF), jnp.float32),
            pltpu.SemaphoreType.DMA,
        ],
    )


def _gather_body(x_hbm, src_hbm, out_hbm, idx_v, rows_v, sem):
    wid = lax.axis_index("s") * NC + lax.axis_index("c")

    def chunk(off):
        pltpu.sync_copy(src_hbm.at[pl.ds(off, CH)], idx_v)
        pltpu.async_copy(x_hbm.at[idx_v], rows_v, sem).wait()
        pltpu.sync_copy(rows_v, out_hbm.at[pl.ds(off, CH)])

    def body(c, carry):
        chunk((wid * CPT + c) * CH)
        return carry

    lax.fori_loop(0, CPT, body, 0)

    @pl.when(wid < EXTRA)
    def _extra():
        chunk((NW * CPT + wid) * CH)


@functools.cache
def _scatter_add_kernel():
    return pl.kernel(
        _scatter_body,
        out_type=jax.ShapeDtypeStruct((2 * N, DF), jnp.float32),
        mesh=_sc_mesh(),
        scratch_types=[
            pltpu.VMEM((1, CH), jnp.int32),
            pltpu.VMEM((CH, DF), jnp.float32),
            pltpu.VMEM_SHARED((N, DF), jnp.float32),
            pltpu.SemaphoreType.DMA,
        ],
    )


def _scatter_body(msg_hbm, dst_hbm, zero_hbm, out_hbm, idx_v, val_v, acc_sh, sem):
    cid = lax.axis_index("c")
    sid = lax.axis_index("s")
    wid = sid * NC + cid
    ZR = N // 10

    @pl.when(sid < 10)
    def _zero():
        pltpu.sync_copy(zero_hbm.at[pl.ds(sid * ZR, ZR)],
                        acc_sh.at[pl.ds(sid * ZR, ZR)])

    plsc.subcore_barrier()

    def chunk(off):
        pltpu.sync_copy(dst_hbm.at[pl.ds(off, CH)], idx_v.at[0])
        pltpu.sync_copy(msg_hbm.at[pl.ds(off, CH)], val_v)
        pltpu.async_copy(val_v, acc_sh.at[idx_v.at[0]], sem, add=True).wait()

    def body(c, carry):
        chunk((wid * CPT + c) * CH)
        return carry

    lax.fori_loop(0, CPT, body, 0)

    @pl.when(wid < EXTRA)
    def _extra():
        chunk((NW * CPT + wid) * CH)

    plsc.subcore_barrier()

    @pl.when(sid < 10)
    def _flush():
        pltpu.sync_copy(acc_sh.at[pl.ds(sid * ZR, ZR)],
                        out_hbm.at[pl.ds(cid * N + sid * ZR, ZR)])


B = 2000


def _msg_body(ea_ref, xs_ref, w_ref, b_ref, out_ref):
    ea = ea_ref[...]
    xs = xs_ref[...]
    lin = jnp.broadcast_to(b_ref[...], (B, OC * DF))
    for d in range(DE):
        lin = lin + ea[:, d:d + 1] * w_ref[d:d + 1, :]
    t = jnp.maximum(lin, 0.0)
    prod = t * jnp.concatenate([xs] * OC, axis=1)
    parts = [jnp.sum(prod[:, o * DF:(o + 1) * DF], axis=1, keepdims=True)
             for o in range(OC)]
    parts.append(jnp.zeros((B, DF - OC), jnp.float32))
    out_ref[...] = jnp.concatenate(parts, axis=1)


def _messages(edge_attr, xs, W1p, b1p):
    return pl.pallas_call(
        _msg_body,
        grid=(EH // B,),
        in_specs=[
            pl.BlockSpec((B, DE), lambda i: (i, 0)),
            pl.BlockSpec((B, DF), lambda i: (i, 0)),
            pl.BlockSpec((DE, OC * DF), lambda i: (0, 0)),
            pl.BlockSpec((1, OC * DF), lambda i: (0, 0)),
        ],
        out_specs=pl.BlockSpec((B, DF), lambda i: (i, 0)),
        out_shape=jax.ShapeDtypeStruct((EH, DF), jnp.float32),
        compiler_params=pltpu.CompilerParams(
            dimension_semantics=("parallel",)),
    )(edge_attr, xs, W1p, b1p)


CK = 2000


def _tail_body(a2_ref, b2_ref, x_ref, root_ref, bc_ref, gm_ref, bt_ref,
               batch_ref, wo_ref, bo_ref, out_ref):
    aggr = (a2_ref[0:N, 0:OC] + a2_ref[N:2 * N, 0:OC]
            + b2_ref[0:N, 0:OC] + b2_ref[N:2 * N, 0:OC])
    h = aggr + jnp.dot(x_ref[...], root_ref[...],
                       precision=lax.Precision.HIGHEST,
                       preferred_element_type=jnp.float32) + bc_ref[...]
    h = jnp.maximum(h, 0.0)
    mean = jnp.mean(h, axis=0, keepdims=True)
    var = jnp.mean(h * h, axis=0, keepdims=True) - mean * mean
    hn = gm_ref[...] * (h - mean) * lax.rsqrt(var + 1e-5) + bt_ref[...]
    gids = lax.broadcasted_iota(jnp.int32, (G, 1), 0)
    pooled = jnp.zeros((G, OC), jnp.float32)
    for k in range(N // CK):
        bk = batch_ref[0:1, k * CK:(k + 1) * CK]
        oh = (jnp.broadcast_to(bk, (G, CK)) == gids).astype(jnp.float32)
        pooled = pooled + lax.dot_general(
            oh, hn[k * CK:(k + 1) * CK, :],
            (((1,), (0,)), ((), ())), precision=lax.Precision.HIGHEST,
            preferred_element_type=jnp.float32)
    pr = jnp.maximum(pooled, 0.0)
    out_ref[...] = jnp.sum(pr * wo_ref[...], axis=1, keepdims=True) + bo_ref[...]


def _tail(a2, b2, x, root, bias_conv, gamma, beta, batch_row, W_out_row, b_out):
    return pl.pallas_call(
        _tail_body,
        out_shape=jax.ShapeDtypeStruct((G, 1), jnp.float32),
    )(a2, b2, x, root, bias_conv, gamma, beta, batch_row, W_out_row, b_out)


@jax.jit
def kernel(x, edge_index, edge_attr, batch, W1, b1, root, bias_conv,
           gamma, beta, W_out, b_out):
    src = edge_index[0].astype(jnp.int32)
    dst = edge_index[1].astype(jnp.int32)
    W1p = W1.reshape(DE, DF, OC).transpose(0, 2, 1).reshape(DE, OC * DF)
    b1p = b1.reshape(DF, OC).T.reshape(1, OC * DF)
    zeros = jnp.zeros((N, DF), jnp.float32)

    gather = _gather_rows_kernel()
    scatter = _scatter_add_kernel()
    xsA = gather(x, src[:EH])
    xsB = gather(x, src[EH:])
    msgA = _messages(edge_attr[:EH], xsA, W1p, b1p)
    a2A = scatter(msgA, dst[:EH], zeros)
    msgB = _messages(edge_attr[EH:], xsB, W1p, b1p)
    a2B = scatter(msgB, dst[EH:], zeros)
    out = _tail(a2A, a2B, x, root,
                bias_conv.reshape(1, OC), gamma.reshape(1, OC),
                beta.reshape(1, OC),
                batch.astype(jnp.int32).reshape(1, N),
                W_out.reshape(1, OC), b_out.reshape(1, 1))
    return out
